# baseline (speedup 1.0000x reference)
import functools

import jax
import jax.numpy as jnp
from jax import lax
from jax.experimental import pallas as pl
from jax.experimental.pallas import tpu as pltpu
from jax.experimental.pallas import tpu_sc as plsc

_LANES = 16
_NUM_WORKERS = 32
_NB = 128
_JB = 112
_RING = 3
_TILE = 128


def _src_col(j, s):
    return j + j // s + 1


def _body(n_rows, in_cols, out_cols, s,
          arr_hbm, out_hbm, inb, outb0, outb1, ti0, ti1, ti2, so0, so1):
    c = lax.axis_index("c")
    sub = lax.axis_index("s")
    wid = sub * 2 + c
    n0 = pl.multiple_of(wid * _NB, _TILE)
    outbs = (outb0, outb1)
    tile_sems = (ti0, ti1, ti2)
    out_sems = (so0, so1)
    iota = lax.iota(jnp.int32, _LANES)
    row_idx = [iota + _LANES * q for q in range(_NB // _LANES)]
    n_batches = out_cols // _JB

    def fetch_tile(t):
        slot = t % _RING
        return pltpu.make_async_copy(
            arr_hbm.at[pl.ds(n0, _NB), pl.ds(t * _TILE, _TILE)],
            inb.at[:, pl.ds((t % _RING) * _TILE, _TILE)],
            tile_sems[slot])

    def out_cp(b, slot):
        return pltpu.make_async_copy(
            outbs[slot], out_hbm.at[pl.ds(b * _JB, _JB), pl.ds(n0, _NB)],
            out_sems[slot])

    ct0 = [_src_col(b * _JB, s) // _TILE for b in range(n_batches)]
    ct1 = [_src_col((b + 1) * _JB - 1, s) // _TILE for b in range(n_batches)]
    first_use = {}
    last_use = {}
    for b in range(n_batches):
        for t in range(ct0[b], ct1[b] + 1):
            first_use.setdefault(t, b)
            last_use[t] = b
    for t in first_use:
        if t + _RING in first_use:
            assert last_use[t] < first_use[t + _RING] - 1, (t, last_use[t])

    for t in range(ct0[0], ct1[0] + 1):
        fetch_tile(t).start()
    hi = ct1[0]

    def compact(b, dst):
        @plsc.parallel_loop(0, _JB, unroll=4)
        def _(j):
            jj = b * _JB + j
            col = jj + jj // s + 1
            ct = col // _TILE
            cl = (ct % _RING) * _TILE + (col % _TILE)
            cv = iota * 0 + cl
            for q in range(_NB // _LANES):
                x = plsc.load_gather(inb, [row_idx[q], cv])
                dst[j, pl.ds(_LANES * q, _LANES)] = x

    waited = ct0[0] - 1
    for b in range(n_batches):
        slot = b % 2
        if b + 1 < n_batches and ct1[b + 1] > hi:
            for t in range(hi + 1, ct1[b + 1] + 1):
                fetch_tile(t).start()
            hi = ct1[b + 1]
        for t in range(waited + 1, ct1[b] + 1):
            fetch_tile(t).wait()
        waited = max(waited, ct1[b])
        if b >= 2:
            out_cp(b - 2, slot).wait()
        compact(b, outbs[slot])
        out_cp(b, slot).start()

    for b in (n_batches - 2, n_batches - 1):
        out_cp(b, b % 2).wait()


def kernel(arr, S):
    del S
    n_rows, in_cols = arr.shape
    s = int(round(in_cols ** 0.5))
    out_cols = s * (s - 1)
    assert s % _LANES == 0
    assert n_rows == _NUM_WORKERS * _NB
    assert out_cols % _JB == 0 and _JB % 8 == 0

    mesh = plsc.VectorSubcoreMesh(core_axis_name="c", subcore_axis_name="s")
    f = pl.kernel(
        functools.partial(_body, n_rows, in_cols, out_cols, s),
        out_type=jax.ShapeDtypeStruct((out_cols, n_rows), jnp.float32),
        mesh=mesh,
        scratch_types=[
            pltpu.VMEM((_NB, _RING * _TILE), jnp.float32),
            pltpu.VMEM((_JB, _NB), jnp.float32),
            pltpu.VMEM((_JB, _NB), jnp.float32),
            pltpu.SemaphoreType.DMA,
            pltpu.SemaphoreType.DMA,
            pltpu.SemaphoreType.DMA,
            pltpu.SemaphoreType.DMA,
            pltpu.SemaphoreType.DMA,
        ],
        compiler_params=pltpu.CompilerParams(needs_layout_passes=False,
                                             use_tc_tiling_on_sc=True),
    )
    return f(arr).T


# speedup vs baseline: 1.8849x; 1.8849x over previous
import functools

import jax
import jax.numpy as jnp
from jax import lax
from jax.experimental import pallas as pl
from jax.experimental.pallas import tpu as pltpu
from jax.experimental.pallas import tpu_sc as plsc

_LANES = 16
_NUM_WORKERS = 32
_RB = 8
_NBUF = 2


def _body(n_rows, in_cols, out_cols, s,
          arr_hbm, out_hbm, inb0, inb1, outb0, outb1, si0, si1, so0, so1):
    c = lax.axis_index("c")
    sub = lax.axis_index("s")
    wid = sub * 2 + c
    rows_per_w = n_rows // _NUM_WORKERS
    base_row = wid * rows_per_w
    n_batches = rows_per_w // _RB
    inbs = (inb0, inb1)
    outbs = (outb0, outb1)
    in_sems = (si0, si1)
    out_sems = (so0, so1)
    iota = lax.iota(jnp.int32, _LANES)
    row_idx = [iota * 0 + r for r in range(_RB)]
    g_per_seg = s // _LANES

    def in_cp(i, slot):
        row = base_row + i * _RB
        return pltpu.make_async_copy(
            arr_hbm.at[pl.ds(row, _RB)], inbs[slot], in_sems[slot])

    def out_cp(i, slot):
        row = base_row + i * _RB
        return pltpu.make_async_copy(
            outbs[slot], out_hbm.at[pl.ds(row, _RB)], out_sems[slot])

    def compact(slot):
        src = inbs[slot]
        dst = outbs[slot]

        @plsc.parallel_loop(0, out_cols // _LANES, unroll=8)
        def _(g):
            col = iota + (_LANES * g + g // g_per_seg + 1)
            for r in range(_RB):
                x = plsc.load_gather(src, [row_idx[r], col])
                dst[r, pl.ds(_LANES * g, _LANES)] = x

    for slot in range(_NBUF):
        in_cp(slot, slot).start()

    def step(k, carry):
        for slot in range(_NBUF):
            i = _NBUF * k + slot
            in_cp(i, slot).wait()

            @pl.when(k >= 1)
            def _():
                out_cp(i - _NBUF, slot).wait()

            compact(slot)
            out_cp(i, slot).start()

            @pl.when(k <= n_batches // _NBUF - 2)
            def _():
                in_cp(i + _NBUF, slot).start()
        return carry

    lax.fori_loop(0, n_batches // _NBUF, step, 0)

    for slot in range(_NBUF):
        out_cp(n_batches - _NBUF + slot, slot).wait()


def kernel(arr, S):
    del S
    n_rows, in_cols = arr.shape
    s = int(round(in_cols ** 0.5))
    out_cols = s * (s - 1)
    assert s % _LANES == 0
    assert n_rows % (_NUM_WORKERS * _RB * _NBUF) == 0

    mesh = plsc.VectorSubcoreMesh(core_axis_name="c", subcore_axis_name="s")
    f = pl.kernel(
        functools.partial(_body, n_rows, in_cols, out_cols, s),
        out_type=jax.ShapeDtypeStruct((n_rows, out_cols), jnp.float32),
        mesh=mesh,
        scratch_types=[
            pltpu.VMEM((_RB, in_cols), jnp.float32),
            pltpu.VMEM((_RB, in_cols), jnp.float32),
            pltpu.VMEM((_RB, out_cols), jnp.float32),
            pltpu.VMEM((_RB, out_cols), jnp.float32),
            pltpu.SemaphoreType.DMA,
            pltpu.SemaphoreType.DMA,
            pltpu.SemaphoreType.DMA,
            pltpu.SemaphoreType.DMA,
        ],
        compiler_params=pltpu.CompilerParams(needs_layout_passes=False,
                                             use_tc_tiling_on_sc=True,
                                             skip_device_barrier=True),
    )
    return f(arr)
